# baseline (speedup 1.0000x reference)
import functools

import jax
import jax.numpy as jnp
from jax import lax
from jax.experimental import pallas as pl
from jax.experimental.pallas import tpu as pltpu
from jax.experimental.pallas import tpu_sc as plsc

N = 10000
D = 128
E = 320000
NC = 2
NS = 16
NW = NC * NS
EPT = E // NW
K = 80
CH = EPT // K
ACC = 10112
SROWS = ACC // NS
ZCP = SROWS // K
ZREM = SROWS % K

_mesh = plsc.VectorSubcoreMesh(core_axis_name="c", subcore_axis_name="s")


def _deg_body(edges_hbm, out_hbm, ones_v, stripe_v, didx, acc, s0, s1, s2, s3):
    sems = (s0, s1, s2, s3)
    c = lax.axis_index("c")
    s = lax.axis_index("s")
    tid = c * NS + s

    for t in range(K // 16):
        ones_v[pl.ds(t * 16, 16)] = jnp.zeros((16,), jnp.float32)
    for t in range(ZCP):
        pltpu.sync_copy(ones_v, acc.at[pl.ds(s * SROWS + t * K, K)])
    pltpu.sync_copy(ones_v.at[pl.ds(0, ZREM)],
                    acc.at[pl.ds(s * SROWS + ZCP * K, ZREM)])

    for t in range(K // 16):
        ones_v[pl.ds(t * 16, 16)] = jnp.ones((16,), jnp.float32)
    plsc.subcore_barrier()

    pltpu.sync_copy(edges_hbm.at[pl.ds(E + tid * EPT, EPT)], didx)

    for b in range(4):
        pltpu.async_copy(ones_v, acc.at[didx.at[pl.ds(b * K, K)]], sems[b], add=True)

    def _step(i, carry):
        for b in range(4):
            j = 4 * i + b
            pltpu.make_async_copy(ones_v, acc.at[didx.at[pl.ds((j - 4) * K, K)]],
                                  sems[b]).wait()

            @pl.when(j < CH)
            def _():
                pltpu.async_copy(ones_v, acc.at[didx.at[pl.ds(j * K, K)]], sems[b],
                                 add=True)

        return carry

    lax.fori_loop(1, (CH + 3) // 4, _step, 0)
    for j in range(4 * ((CH + 3) // 4 - 1), CH):
        pltpu.make_async_copy(ones_v, acc.at[didx.at[pl.ds(j * K, K)]], sems[j % 4]).wait()
    plsc.subcore_barrier()
    pltpu.sync_copy(acc.at[pl.ds(s * SROWS, SROWS)], stripe_v)
    pltpu.sync_copy(stripe_v, out_hbm.at[pl.ds(c * ACC + s * SROWS, SROWS)])


_deg_kernel = pl.kernel(
    _deg_body,
    out_type=jax.ShapeDtypeStruct((NC * ACC,), jnp.float32),
    mesh=_mesh,
    scratch_types=[
        pltpu.VMEM((K,), jnp.float32),
        pltpu.VMEM((SROWS,), jnp.float32),
        pltpu.VMEM((EPT,), jnp.int32),
        pltpu.VMEM_SHARED((ACC,), jnp.float32),
        pltpu.SemaphoreType.DMA,
        pltpu.SemaphoreType.DMA,
        pltpu.SemaphoreType.DMA,
        pltpu.SemaphoreType.DMA,
    ],
)


def _seg_body(h_hbm, edges_hbm, out_hbm, sidx, didx,
              rba, rbb, acc, ga, gb, sa, sb):
    c = lax.axis_index("c")
    s = lax.axis_index("s")
    tid = c * NS + s

    def _zero(r, carry):
        for t in range(D // 16):
            rba[r, pl.ds(t * 16, 16)] = jnp.zeros((16,), jnp.float32)
        return carry

    lax.fori_loop(0, K, _zero, 0)
    for t in range(ZCP):
        pltpu.sync_copy(rba, acc.at[pl.ds(s * SROWS + t * K, K)])
    pltpu.sync_copy(rba.at[pl.ds(0, ZREM)],
                    acc.at[pl.ds(s * SROWS + ZCP * K, ZREM)])
    plsc.subcore_barrier()

    pltpu.sync_copy(edges_hbm.at[pl.ds(tid * EPT, EPT)], sidx)
    pltpu.sync_copy(edges_hbm.at[pl.ds(E + tid * EPT, EPT)], didx)

    def _gidx(j):
        return sidx.at[pl.ds(j * K, K)]

    def _sidx_ref(j):
        return didx.at[pl.ds(j * K, K)]

    pltpu.async_copy(h_hbm.at[_gidx(0)], rba, ga)
    pltpu.async_copy(h_hbm.at[_gidx(1)], rbb, gb)

    def _step(i, carry):
        ja = 2 * i
        jb = 2 * i + 1
        pltpu.make_async_copy(h_hbm.at[_gidx(ja)], rba, ga).wait()
        pltpu.async_copy(rba, acc.at[_sidx_ref(ja)], sa, add=True)
        pltpu.make_async_copy(h_hbm.at[_gidx(jb)], rbb, gb).wait()
        pltpu.async_copy(rbb, acc.at[_sidx_ref(jb)], sb, add=True)
        pltpu.make_async_copy(rba, acc.at[_sidx_ref(ja)], sa).wait()
        pltpu.async_copy(h_hbm.at[_gidx(ja + 2)], rba, ga)
        jc = jnp.minimum(jb + 2, CH - 1)
        pltpu.make_async_copy(rbb, acc.at[_sidx_ref(jb)], sb).wait()
        pltpu.async_copy(h_hbm.at[_gidx(jc)], rbb, gb)
        return carry

    lax.fori_loop(0, CH // 2, _step, 0)
    pltpu.make_async_copy(h_hbm.at[_gidx(CH - 1)], rba, ga).wait()
    pltpu.sync_copy(rba, acc.at[_sidx_ref(CH - 1)], add=True)
    pltpu.make_async_copy(h_hbm.at[_gidx(CH - 1)], rbb, gb).wait()
    plsc.subcore_barrier()
    pltpu.sync_copy(acc.at[pl.ds(s * SROWS, SROWS)],
                    out_hbm.at[c, pl.ds(s * SROWS, SROWS)])


_seg_kernel = pl.kernel(
    _seg_body,
    out_type=jax.ShapeDtypeStruct((NC, ACC, D), jnp.float32),
    mesh=_mesh,
    scratch_types=[
        pltpu.VMEM((EPT,), jnp.int32),
        pltpu.VMEM((EPT,), jnp.int32),
        pltpu.VMEM((K, D), jnp.float32),
        pltpu.VMEM((K, D), jnp.float32),
        pltpu.VMEM_SHARED((ACC, D), jnp.float32),
        pltpu.SemaphoreType.DMA,
        pltpu.SemaphoreType.DMA,
        pltpu.SemaphoreType.DMA,
        pltpu.SemaphoreType.DMA,
    ],
)


_BLK = 1024
_GRID = (N + _BLK - 1) // _BLK


def _dis_block(dp_b):
    deg = dp_b[0] + dp_b[1] + 1.0
    return lax.rsqrt(deg)[:, None]


def _tc1_body(x_b, w1_b, dp_b, o_b):
    o_b[...] = jnp.dot(x_b[...] * _dis_block(dp_b), w1_b[...],
                       preferred_element_type=jnp.float32)


def _tc2_body(p_b, h1p_b, dp_b, b1_b, w2_b, o_b):
    dis = _dis_block(dp_b)
    sfull = p_b[0] + p_b[1] + h1p_b[...]
    z = jnp.maximum(dis * sfull + b1_b[...], 0.0)
    o_b[...] = jnp.dot(z, w2_b[...], preferred_element_type=jnp.float32) * dis


def _tc3_body(q_b, h2p_b, dp_b, b2_b, o_b):
    dis = _dis_block(dp_b)
    o_b[...] = dis * (q_b[0] + q_b[1] + h2p_b[...]) + b2_b[...]


def _dis_spec():
    return pl.BlockSpec((NC, _BLK), lambda r: (0, r))


def _part_spec():
    return pl.BlockSpec((NC, _BLK, D), lambda r: (0, r, 0))


def _row_spec():
    return pl.BlockSpec((_BLK, D), lambda r: (r, 0))


def _full_spec(rows):
    return pl.BlockSpec((rows, D), lambda r: (0, 0))


_tc1 = pl.pallas_call(
    _tc1_body,
    grid=(_GRID,),
    in_specs=[_row_spec(), _full_spec(D), _dis_spec()],
    out_specs=_row_spec(),
    out_shape=jax.ShapeDtypeStruct((N, D), jnp.float32),
)

_tc2 = pl.pallas_call(
    _tc2_body,
    grid=(_GRID,),
    in_specs=[_part_spec(), _row_spec(), _dis_spec(), _full_spec(1),
              _full_spec(D)],
    out_specs=_row_spec(),
    out_shape=jax.ShapeDtypeStruct((N, D), jnp.float32),
)

_tc3 = pl.pallas_call(
    _tc3_body,
    grid=(_GRID,),
    in_specs=[_part_spec(), _row_spec(), _dis_spec(), _full_spec(1)],
    out_specs=_row_spec(),
    out_shape=jax.ShapeDtypeStruct((N, D), jnp.float32),
)


def kernel(x, edge_index, W1, b1, W2, b2):
    ei = edge_index.astype(jnp.int32)
    eflat = ei.reshape(2 * E)
    b1r = b1.reshape(1, D)
    b2r = b2.reshape(1, D)

    dp = _deg_kernel(eflat).reshape(NC, ACC)
    h1p = _tc1(x, W1, dp)
    p1 = _seg_kernel(h1p, eflat)
    h2p = _tc2(p1, h1p, dp, b1r, W2)
    p2 = _seg_kernel(h2p, eflat)
    return _tc3(p2, h2p, dp, b2r)


# speedup vs baseline: 1.2446x; 1.2446x over previous
import functools

import jax
import jax.numpy as jnp
from jax import lax
from jax.experimental import pallas as pl
from jax.experimental.pallas import tpu as pltpu
from jax.experimental.pallas import tpu_sc as plsc

N = 10000
D = 128
E = 320000
NC = 2
NS = 16
NW = NC * NS
EPT = E // NW
K = 80
CH = EPT // K
ACC = 10112
SROWS = ACC // NS
ZCP = SROWS // K
ZREM = SROWS % K

_mesh = plsc.VectorSubcoreMesh(core_axis_name="c", subcore_axis_name="s")


def _deg_body(edges_hbm, out_hbm, ones_v, stripe_v, didx, acc, s0, s1, s2, s3):
    sems = (s0, s1, s2, s3)
    c = lax.axis_index("c")
    s = lax.axis_index("s")
    tid = c * NS + s

    for t in range(K // 16):
        ones_v[pl.ds(t * 16, 16)] = jnp.zeros((16,), jnp.float32)
    for t in range(ZCP):
        pltpu.sync_copy(ones_v, acc.at[pl.ds(s * SROWS + t * K, K)])
    pltpu.sync_copy(ones_v.at[pl.ds(0, ZREM)],
                    acc.at[pl.ds(s * SROWS + ZCP * K, ZREM)])

    for t in range(K // 16):
        ones_v[pl.ds(t * 16, 16)] = jnp.ones((16,), jnp.float32)
    plsc.subcore_barrier()

    pltpu.sync_copy(edges_hbm.at[pl.ds(E + tid * EPT, EPT)], didx)

    for b in range(4):
        pltpu.async_copy(ones_v, acc.at[didx.at[pl.ds(b * K, K)]], sems[b], add=True)

    def _step(i, carry):
        for b in range(4):
            j = 4 * i + b
            pltpu.make_async_copy(ones_v, acc.at[didx.at[pl.ds((j - 4) * K, K)]],
                                  sems[b]).wait()

            @pl.when(j < CH)
            def _():
                pltpu.async_copy(ones_v, acc.at[didx.at[pl.ds(j * K, K)]], sems[b],
                                 add=True)

        return carry

    lax.fori_loop(1, (CH + 3) // 4, _step, 0)
    for j in range(4 * ((CH + 3) // 4 - 1), CH):
        pltpu.make_async_copy(ones_v, acc.at[didx.at[pl.ds(j * K, K)]], sems[j % 4]).wait()
    plsc.subcore_barrier()
    pltpu.sync_copy(acc.at[pl.ds(s * SROWS, SROWS)], stripe_v)
    pltpu.sync_copy(stripe_v, out_hbm.at[pl.ds(c * ACC + s * SROWS, SROWS)])


_deg_kernel = pl.kernel(
    _deg_body,
    out_type=jax.ShapeDtypeStruct((NC * ACC,), jnp.float32),
    mesh=_mesh,
    scratch_types=[
        pltpu.VMEM((K,), jnp.float32),
        pltpu.VMEM((SROWS,), jnp.float32),
        pltpu.VMEM((EPT,), jnp.int32),
        pltpu.VMEM_SHARED((ACC,), jnp.float32),
        pltpu.SemaphoreType.DMA,
        pltpu.SemaphoreType.DMA,
        pltpu.SemaphoreType.DMA,
        pltpu.SemaphoreType.DMA,
    ],
)


def _seg_body(h_hbm, edges_hbm, out_hbm, sidx, didx,
              rba, rbb, acc, ga, gb):
    c = lax.axis_index("c")
    s = lax.axis_index("s")
    tid = c * NS + s

    def _zero(r, carry):
        for t in range(D // 16):
            rba[r, pl.ds(t * 16, 16)] = jnp.zeros((16,), jnp.float32)
        return carry

    lax.fori_loop(0, K, _zero, 0)
    for t in range(ZCP):
        pltpu.sync_copy(rba, acc.at[pl.ds(s * SROWS + t * K, K)])
    pltpu.sync_copy(rba.at[pl.ds(0, ZREM)],
                    acc.at[pl.ds(s * SROWS + ZCP * K, ZREM)])
    plsc.subcore_barrier()

    pltpu.sync_copy(edges_hbm.at[pl.ds(tid * EPT, EPT)], sidx)
    pltpu.sync_copy(edges_hbm.at[pl.ds(E + tid * EPT, EPT)], didx)

    def _gidx(j):
        return sidx.at[pl.ds(j * K, K)]

    def _sidx_ref(j):
        return didx.at[pl.ds(j * K, K)]

    pltpu.async_copy(h_hbm.at[_gidx(0)], rba, ga)

    def _step(i, carry):
        ja = 2 * i
        jb = 2 * i + 1
        pltpu.async_copy(h_hbm.at[_gidx(jb)], rbb, gb)
        pltpu.make_async_copy(h_hbm.at[_gidx(ja)], rba, ga).wait()
        pltpu.sync_copy(rba, acc.at[_sidx_ref(ja)], add=True)
        pltpu.async_copy(h_hbm.at[_gidx(jb + 1)], rba, ga)
        pltpu.make_async_copy(h_hbm.at[_gidx(jb)], rbb, gb).wait()
        pltpu.sync_copy(rbb, acc.at[_sidx_ref(jb)], add=True)
        return carry

    lax.fori_loop(0, CH // 2, _step, 0)
    pltpu.make_async_copy(h_hbm.at[_gidx(CH - 1)], rba, ga).wait()
    pltpu.sync_copy(rba, acc.at[_sidx_ref(CH - 1)], add=True)
    plsc.subcore_barrier()
    pltpu.sync_copy(acc.at[pl.ds(s * SROWS, SROWS)],
                    out_hbm.at[c, pl.ds(s * SROWS, SROWS)])


_seg_kernel = pl.kernel(
    _seg_body,
    out_type=jax.ShapeDtypeStruct((NC, ACC, D), jnp.float32),
    mesh=_mesh,
    scratch_types=[
        pltpu.VMEM((EPT,), jnp.int32),
        pltpu.VMEM((EPT,), jnp.int32),
        pltpu.VMEM((K, D), jnp.float32),
        pltpu.VMEM((K, D), jnp.float32),
        pltpu.VMEM_SHARED((ACC, D), jnp.float32),
        pltpu.SemaphoreType.DMA,
        pltpu.SemaphoreType.DMA,
    ],
)


_BLK = 1024
_GRID = (N + _BLK - 1) // _BLK


def _dis_block(dp_b):
    deg = dp_b[0] + dp_b[1] + 1.0
    return lax.rsqrt(deg)[:, None]


def _tc1_body(x_b, w1_b, dp_b, o_b):
    o_b[...] = jnp.dot(x_b[...] * _dis_block(dp_b), w1_b[...],
                       preferred_element_type=jnp.float32)


def _tc2_body(p_b, h1p_b, dp_b, b1_b, w2_b, o_b):
    dis = _dis_block(dp_b)
    sfull = p_b[0] + p_b[1] + h1p_b[...]
    z = jnp.maximum(dis * sfull + b1_b[...], 0.0)
    o_b[...] = jnp.dot(z, w2_b[...], preferred_element_type=jnp.float32) * dis


def _tc3_body(q_b, h2p_b, dp_b, b2_b, o_b):
    dis = _dis_block(dp_b)
    o_b[...] = dis * (q_b[0] + q_b[1] + h2p_b[...]) + b2_b[...]


def _dis_spec():
    return pl.BlockSpec((NC, _BLK), lambda r: (0, r))


def _part_spec():
    return pl.BlockSpec((NC, _BLK, D), lambda r: (0, r, 0))


def _row_spec():
    return pl.BlockSpec((_BLK, D), lambda r: (r, 0))


def _full_spec(rows):
    return pl.BlockSpec((rows, D), lambda r: (0, 0))


_tc1 = pl.pallas_call(
    _tc1_body,
    grid=(_GRID,),
    in_specs=[_row_spec(), _full_spec(D), _dis_spec()],
    out_specs=_row_spec(),
    out_shape=jax.ShapeDtypeStruct((N, D), jnp.float32),
)

_tc2 = pl.pallas_call(
    _tc2_body,
    grid=(_GRID,),
    in_specs=[_part_spec(), _row_spec(), _dis_spec(), _full_spec(1),
              _full_spec(D)],
    out_specs=_row_spec(),
    out_shape=jax.ShapeDtypeStruct((N, D), jnp.float32),
)

_tc3 = pl.pallas_call(
    _tc3_body,
    grid=(_GRID,),
    in_specs=[_part_spec(), _row_spec(), _dis_spec(), _full_spec(1)],
    out_specs=_row_spec(),
    out_shape=jax.ShapeDtypeStruct((N, D), jnp.float32),
)


def kernel(x, edge_index, W1, b1, W2, b2):
    ei = edge_index.astype(jnp.int32)
    eflat = ei.reshape(2 * E)
    b1r = b1.reshape(1, D)
    b2r = b2.reshape(1, D)

    dp = _deg_kernel(eflat).reshape(NC, ACC)
    h1p = _tc1(x, W1, dp)
    p1 = _seg_kernel(h1p, eflat)
    h2p = _tc2(p1, h1p, dp, b1r, W2)
    p2 = _seg_kernel(h2p, eflat)
    return _tc3(p2, h2p, dp, b2r)
